# Initial kernel scaffold; baseline (speedup 1.0000x reference)
import functools

import jax
import jax.numpy as jnp
from jax import lax
from jax.experimental import pallas as pl
from jax.experimental.pallas import tpu as pltpu
from jax.experimental.pallas import tpu_sc as plsc

N = 10002
D = 128
HALF = 64
N_PAD = 10240
RPT = 640
C = 128
NTILES = 16
NCORES = 2
VPR = HALF // 16


def _body(final, nchunks, ept, *refs):
    if final:
        (x_hbm, cols_hbm, rows_hbm, vals_hbm, x0_hbm, x1_hbm, out_hbm,
         acc, cols_v, rows_v, vals_v, gbuf, zbuf, xb0, xb1, sem) = refs
    else:
        (x_hbm, cols_hbm, rows_hbm, vals_hbm, out_hbm,
         acc, cols_v, rows_v, vals_v, gbuf, zbuf, sem) = refs

    c = lax.axis_index("c")
    s = lax.axis_index("s")
    e_pad = nchunks * C * NTILES

    def zrow(i, carry):
        for j in range(VPR):
            zbuf[i, pl.ds(j * 16, 16)] = jnp.zeros((16,), jnp.float32)
        return carry
    lax.fori_loop(0, C, zrow, 0)
    rbase = s * RPT
    for b in range(RPT // C):
        pltpu.sync_copy(zbuf, acc.at[pl.ds(rbase + b * C, C)])
    plsc.subcore_barrier()

    ebase = s * ept
    cbase = c * e_pad + ebase

    def chunk(k, carry):
        off = k * C
        pltpu.sync_copy(cols_hbm.at[pl.ds(cbase + off, C)], cols_v)
        pltpu.sync_copy(rows_hbm.at[pl.ds(ebase + off, C)], rows_v)
        pltpu.sync_copy(vals_hbm.at[pl.ds(ebase + off, C)], vals_v)
        pltpu.async_copy(x_hbm.at[cols_v], gbuf, sem).wait()

        def group(g, carry2):
            vv = vals_v[pl.ds(g * 16, 16)]
            for l in range(16):
                e = g * 16 + l
                sval = vv[l]
                for j in range(VPR):
                    sl = pl.ds(j * 16, 16)
                    gbuf[e, sl] = gbuf[e, sl] * sval
            return carry2
        lax.fori_loop(0, C // 16, group, 0)
        pltpu.sync_copy(gbuf, acc.at[rows_v], add=True)
        return carry
    lax.fori_loop(0, nchunks, chunk, 0)

    plsc.subcore_barrier()

    obase = c * N_PAD + rbase
    for b in range(RPT // C):
        r0 = rbase + b * C
        o0 = obase + b * C
        if not final:
            pltpu.sync_copy(acc.at[pl.ds(r0, C)], out_hbm.at[pl.ds(o0, C)])
        else:
            pltpu.sync_copy(acc.at[pl.ds(r0, C)], gbuf)
            pltpu.sync_copy(x0_hbm.at[pl.ds(o0, C)], xb0)
            pltpu.sync_copy(x1_hbm.at[pl.ds(o0, C)], xb1)

            def crow(i, carry):
                for j in range(VPR):
                    sl = pl.ds(j * 16, 16)
                    gbuf[i, sl] = (gbuf[i, sl] + xb0[i, sl] + xb1[i, sl]) * (
                        1.0 / 3.0)
                return carry
            lax.fori_loop(0, C, crow, 0)
            pltpu.sync_copy(gbuf, out_hbm.at[pl.ds(o0, C)])


def _make_kernel(e_pad, final):
    ept = e_pad // NTILES
    nchunks = ept // C
    mesh = plsc.VectorSubcoreMesh(core_axis_name="c", subcore_axis_name="s")
    scratch = [
        pltpu.VMEM_SHARED((N_PAD, HALF), jnp.float32),
        pltpu.VMEM((C,), jnp.int32),
        pltpu.VMEM((C,), jnp.int32),
        pltpu.VMEM((C,), jnp.float32),
        pltpu.VMEM((C, HALF), jnp.float32),
        pltpu.VMEM((C, HALF), jnp.float32),
    ]
    if final:
        scratch += [
            pltpu.VMEM((C, HALF), jnp.float32),
            pltpu.VMEM((C, HALF), jnp.float32),
        ]
    scratch += [pltpu.SemaphoreType.DMA]
    return pl.kernel(
        functools.partial(_body, final, nchunks, ept),
        out_type=jax.ShapeDtypeStruct((2 * N_PAD, HALF), jnp.float32),
        mesh=mesh,
        scratch_types=scratch,
    )


def kernel(adjacency_indices, adjacency_values, embedding):
    rows = adjacency_indices[0].astype(jnp.int32)
    cols = adjacency_indices[1].astype(jnp.int32)
    vals = adjacency_values.astype(jnp.float32)
    e = vals.shape[0]
    ept = -(-(e // NTILES) // C) * C
    e_pad = ept * NTILES

    cols_p = jnp.pad(cols, (0, e_pad - e))
    rows_p = jnp.pad(rows, (0, e_pad - e), constant_values=N)
    vals_p = jnp.pad(vals, (0, e_pad - e))
    cols2 = jnp.concatenate([cols_p, cols_p + N_PAD])

    emb_pad = jnp.pad(embedding.astype(jnp.float32),
                      ((0, N_PAD - N), (0, 0)))
    x0s = jnp.concatenate([emb_pad[:, :HALF], emb_pad[:, HALF:]], axis=0)

    layer_k = _make_kernel(e_pad, final=False)
    final_k = _make_kernel(e_pad, final=True)

    x1s = layer_k(x0s, cols2, rows_p, vals_p)
    outs = final_k(x1s, cols2, rows_p, vals_p, x0s, x1s)

    full = jnp.concatenate([outs[:N], outs[N_PAD:N_PAD + N]], axis=1)
    ds3 = N // 3
    return jnp.concatenate(
        [full[:ds3], full[ds3:2 * ds3], full[2 * ds3:]], axis=1)


# speedup vs baseline: 2.1972x; 2.1972x over previous
import functools

import jax
import jax.numpy as jnp
from jax import lax
from jax.experimental import pallas as pl
from jax.experimental.pallas import tpu as pltpu
from jax.experimental.pallas import tpu_sc as plsc

N = 10002
D = 128
HALF = 64
N_PAD = 10240
RPT = 640
C = 128
NTILES = 16
NCORES = 2
VPR = HALF // 16


def _body(final, nchunks, ept, *refs):
    if final:
        (x_hbm, cols_hbm, rows_hbm, vals_hbm, x0_hbm, x1_hbm, out_hbm,
         acc, cols_v, rows_v, vals_v, gbuf, zbuf, xb0, xb1, sem) = refs
    else:
        (x_hbm, cols_hbm, rows_hbm, vals_hbm, out_hbm,
         acc, cols_v, rows_v, vals_v, gbuf, zbuf, sem) = refs

    c = lax.axis_index("c")
    s = lax.axis_index("s")
    e_pad = nchunks * C * NTILES

    def zrow(i, carry):
        for j in range(VPR):
            zbuf[i, pl.ds(j * 16, 16)] = jnp.zeros((16,), jnp.float32)
        return carry
    lax.fori_loop(0, C, zrow, 0)
    rbase = s * RPT
    for b in range(RPT // C):
        pltpu.sync_copy(zbuf, acc.at[pl.ds(rbase + b * C, C)])
    plsc.subcore_barrier()

    ebase = s * ept
    cbase = c * e_pad + ebase

    def chunk(k, carry):
        off = k * C
        pltpu.sync_copy(cols_hbm.at[pl.ds(cbase + off, C)], cols_v)
        pltpu.sync_copy(rows_hbm.at[pl.ds(ebase + off, C)], rows_v)
        pltpu.sync_copy(vals_hbm.at[pl.ds(ebase + off, C)], vals_v)
        pltpu.async_copy(x_hbm.at[cols_v], gbuf, sem).wait()

        def group(g, carry2):
            vv = vals_v[pl.ds(g * 16, 16)]
            for l in range(16):
                e = g * 16 + l
                sval = vv[l]
                for j in range(VPR):
                    sl = pl.ds(j * 16, 16)
                    gbuf[e, sl] = gbuf[e, sl] * sval
            return carry2
        lax.fori_loop(0, C // 16, group, 0)
        pltpu.sync_copy(gbuf, acc.at[rows_v], add=True)
        return carry
    lax.fori_loop(0, nchunks, chunk, 0)

    plsc.subcore_barrier()

    obase = c * N_PAD + rbase
    for b in range(RPT // C):
        r0 = rbase + b * C
        o0 = obase + b * C
        if not final:
            pltpu.sync_copy(acc.at[pl.ds(r0, C)], out_hbm.at[pl.ds(o0, C)])
        else:
            pltpu.sync_copy(acc.at[pl.ds(r0, C)], gbuf)
            pltpu.sync_copy(x0_hbm.at[pl.ds(o0, C)], xb0)
            pltpu.sync_copy(x1_hbm.at[pl.ds(o0, C)], xb1)

            def crow(i, carry):
                for j in range(VPR):
                    sl = pl.ds(j * 16, 16)
                    gbuf[i, sl] = (gbuf[i, sl] + xb0[i, sl] + xb1[i, sl]) * (
                        1.0 / 3.0)
                return carry
            lax.fori_loop(0, C, crow, 0)
            pltpu.sync_copy(gbuf, out_hbm.at[pl.ds(o0, C)])


def _make_kernel(e_pad, final):
    ept = e_pad // NTILES
    nchunks = ept // C
    mesh = plsc.VectorSubcoreMesh(core_axis_name="c", subcore_axis_name="s")
    scratch = [
        pltpu.VMEM_SHARED((N_PAD, HALF), jnp.float32),
        pltpu.VMEM((C,), jnp.int32),
        pltpu.VMEM((C,), jnp.int32),
        pltpu.VMEM((C,), jnp.float32),
        pltpu.VMEM((C, HALF), jnp.float32),
        pltpu.VMEM((C, HALF), jnp.float32),
    ]
    if final:
        scratch += [
            pltpu.VMEM((C, HALF), jnp.float32),
            pltpu.VMEM((C, HALF), jnp.float32),
        ]
    scratch += [pltpu.SemaphoreType.DMA]
    return pl.kernel(
        functools.partial(_body, final, nchunks, ept),
        out_type=jax.ShapeDtypeStruct((2 * N_PAD, HALF), jnp.float32),
        mesh=mesh,
        scratch_types=scratch,
        compiler_params=pltpu.CompilerParams(use_tc_tiling_on_sc=False),
    )


def kernel(adjacency_indices, adjacency_values, embedding):
    rows = adjacency_indices[0].astype(jnp.int32)
    cols = adjacency_indices[1].astype(jnp.int32)
    vals = adjacency_values.astype(jnp.float32)
    e = vals.shape[0]
    ept = -(-(e // NTILES) // C) * C
    e_pad = ept * NTILES

    cols_p = jnp.pad(cols, (0, e_pad - e))
    rows_p = jnp.pad(rows, (0, e_pad - e), constant_values=N)
    vals_p = jnp.pad(vals, (0, e_pad - e))
    cols2 = jnp.concatenate([cols_p, cols_p + N_PAD])

    emb_pad = jnp.pad(embedding.astype(jnp.float32),
                      ((0, N_PAD - N), (0, 0)))
    x0s = jnp.concatenate([emb_pad[:, :HALF], emb_pad[:, HALF:]], axis=0)

    layer_k = _make_kernel(e_pad, final=False)
    final_k = _make_kernel(e_pad, final=True)

    x1s = layer_k(x0s, cols2, rows_p, vals_p)
    outs = final_k(x1s, cols2, rows_p, vals_p, x0s, x1s)

    full = jnp.concatenate([outs[:N], outs[N_PAD:N_PAD + N]], axis=1)
    ds3 = N // 3
    return jnp.concatenate(
        [full[:ds3], full[ds3:2 * ds3], full[2 * ds3:]], axis=1)


# speedup vs baseline: 5.0685x; 2.3068x over previous
import functools

import jax
import jax.numpy as jnp
from jax import lax
from jax.experimental import pallas as pl
from jax.experimental.pallas import tpu as pltpu
from jax.experimental.pallas import tpu_sc as plsc

N = 10002
D = 128
HALF = 64
N_PAD = 10240
RPT = 640
C = 128
NTILES = 16
NCORES = 2
VPR = HALF // 16
RBITS = 14
RMASK = (1 << RBITS) - 1


def _scale_chunk(gbuf, valsb, k):
    def group(g, carry):
        vv = valsb[k, pl.ds(g * 16, 16)]
        for l in range(16):
            e = g * 16 + l
            sval = vv[l]
            for j in range(VPR):
                sl = pl.ds(j * 16, 16)
                gbuf[e, sl] = gbuf[e, sl] * sval
        return carry
    lax.fori_loop(0, C // 16, group, 0)


def _unpack_chunk(packedb, k, coff, cols_v, rows_v):
    def group(g, carry):
        sl = pl.ds(g * 16, 16)
        p = packedb[k, sl]
        rows_v[sl] = lax.bitwise_and(p, RMASK)
        cols_v[sl] = lax.shift_right_logical(p, RBITS) + coff
        return carry
    lax.fori_loop(0, C // 16, group, 0)


def _body(final, nchunks, *refs):
    if final:
        (x_hbm, packed_hbm, vals_hbm, x0_hbm, x1_hbm, out_hbm,
         acc, packedb, valsb, cv0, rv0, cv1, rv1, gb0, gb1, xb0, xb1,
         sem_g, sem_s) = refs
    else:
        (x_hbm, packed_hbm, vals_hbm, out_hbm,
         acc, packedb, valsb, cv0, rv0, cv1, rv1, gb0, gb1,
         sem_g, sem_s) = refs

    c = lax.axis_index("c")
    s = lax.axis_index("s")
    coff = c * N_PAD

    def zrow(i, carry):
        for j in range(VPR):
            gb0[i, pl.ds(j * 16, 16)] = jnp.zeros((16,), jnp.float32)
        return carry
    lax.fori_loop(0, C, zrow, 0)
    rbase = s * RPT
    for b in range(RPT // C):
        pltpu.sync_copy(gb0, acc.at[pl.ds(rbase + b * C, C)])

    erow0 = s * nchunks
    pltpu.sync_copy(packed_hbm.at[pl.ds(erow0, nchunks)], packedb)
    pltpu.sync_copy(vals_hbm.at[pl.ds(erow0, nchunks)], valsb)
    plsc.subcore_barrier()

    def wait_g():
        pltpu.make_async_copy(x_hbm.at[cv0], gb0, sem_g).wait()

    def wait_s():
        pltpu.make_async_copy(gb0, acc.at[rv0], sem_s).wait()

    n2 = nchunks // 2
    _unpack_chunk(packedb, 0, coff, cv0, rv0)
    pltpu.async_copy(x_hbm.at[cv0], gb0, sem_g)

    def iter_body(i, carry):
        k0 = 2 * i
        k1 = 2 * i + 1
        wait_g()

        @pl.when(i > 0)
        def _():
            wait_s()
        _unpack_chunk(packedb, k1, coff, cv1, rv1)
        pltpu.async_copy(x_hbm.at[cv1], gb1, sem_g)
        _scale_chunk(gb0, valsb, k0)
        pltpu.async_copy(gb0, acc.at[rv0], sem_s, add=True)
        wait_g()
        _scale_chunk(gb1, valsb, k1)
        wait_s()

        @pl.when(i < n2 - 1)
        def _():
            _unpack_chunk(packedb, k0 + 2, coff, cv0, rv0)
            pltpu.async_copy(x_hbm.at[cv0], gb0, sem_g)
        pltpu.async_copy(gb1, acc.at[rv1], sem_s, add=True)
        return carry
    lax.fori_loop(0, n2, iter_body, 0)
    wait_s()

    plsc.subcore_barrier()

    obase = c * N_PAD + rbase
    for b in range(RPT // C):
        r0 = rbase + b * C
        o0 = obase + b * C
        if not final:
            pltpu.sync_copy(acc.at[pl.ds(r0, C)], out_hbm.at[pl.ds(o0, C)])
        else:
            pltpu.sync_copy(acc.at[pl.ds(r0, C)], gb0)
            pltpu.sync_copy(x0_hbm.at[pl.ds(o0, C)], xb0)
            pltpu.sync_copy(x1_hbm.at[pl.ds(o0, C)], xb1)

            def crow(i, carry):
                for j in range(VPR):
                    sl = pl.ds(j * 16, 16)
                    gb0[i, sl] = (gb0[i, sl] + xb0[i, sl] + xb1[i, sl]) * (
                        1.0 / 3.0)
                return carry
            lax.fori_loop(0, C, crow, 0)
            pltpu.sync_copy(gb0, out_hbm.at[pl.ds(o0, C)])


def _make_kernel(nchunks, final):
    mesh = plsc.VectorSubcoreMesh(core_axis_name="c", subcore_axis_name="s")
    scratch = [
        pltpu.VMEM_SHARED((N_PAD, HALF), jnp.float32),
        pltpu.VMEM((nchunks, C), jnp.int32),
        pltpu.VMEM((nchunks, C), jnp.float32),
        pltpu.VMEM((C,), jnp.int32),
        pltpu.VMEM((C,), jnp.int32),
        pltpu.VMEM((C,), jnp.int32),
        pltpu.VMEM((C,), jnp.int32),
        pltpu.VMEM((C, HALF), jnp.float32),
        pltpu.VMEM((C, HALF), jnp.float32),
    ]
    if final:
        scratch += [
            pltpu.VMEM((C, HALF), jnp.float32),
            pltpu.VMEM((C, HALF), jnp.float32),
        ]
    scratch += [pltpu.SemaphoreType.DMA, pltpu.SemaphoreType.DMA]
    return pl.kernel(
        functools.partial(_body, final, nchunks),
        out_type=jax.ShapeDtypeStruct((2 * N_PAD, HALF), jnp.float32),
        mesh=mesh,
        scratch_types=scratch,
        compiler_params=pltpu.CompilerParams(use_tc_tiling_on_sc=False),
    )


def kernel(adjacency_indices, adjacency_values, embedding):
    rows = adjacency_indices[0].astype(jnp.int32)
    cols = adjacency_indices[1].astype(jnp.int32)
    vals = adjacency_values.astype(jnp.float32)
    e = vals.shape[0]
    ept = -(-(e // NTILES) // (2 * C)) * (2 * C)
    nchunks = ept // C
    e_pad = ept * NTILES

    packed = jnp.bitwise_or(rows, jnp.left_shift(cols, RBITS))
    packed_p = jnp.pad(packed, (0, e_pad - e), constant_values=N)
    vals_p = jnp.pad(vals, (0, e_pad - e))
    packed2 = packed_p.reshape(-1, C)
    vals2 = vals_p.reshape(-1, C)

    emb_pad = jnp.pad(embedding.astype(jnp.float32),
                      ((0, N_PAD - N), (0, 0)))
    x0s = jnp.concatenate([emb_pad[:, :HALF], emb_pad[:, HALF:]], axis=0)

    layer_k = _make_kernel(nchunks, final=False)
    final_k = _make_kernel(nchunks, final=True)

    x1s = layer_k(x0s, packed2, vals2)
    outs = final_k(x1s, packed2, vals2, x0s, x1s)

    full = jnp.concatenate([outs[:N], outs[N_PAD:N_PAD + N]], axis=1)
    ds3 = N // 3
    return jnp.concatenate(
        [full[:ds3], full[ds3:2 * ds3], full[2 * ds3:]], axis=1)
